# baseline (speedup 1.0000x reference)
import jax
import jax.numpy as jnp
from jax.experimental import pallas as pl

_N_STATES = 16
_CB = 64


def _gibbs_body(x_ref, w16_ref, g_ref, o_ref):
    xv = x_ref[...]
    n_dims = xv.shape[2]
    logits = w16_ref[...] + g_ref[...]
    m = jnp.max(logits, axis=1, keepdims=True)
    iota = jax.lax.broadcasted_iota(jnp.int32, (_CB, _N_STATES), 1)
    sel = jnp.min(jnp.where(logits == m, iota, _N_STATES), axis=1,
                  keepdims=True)
    onehot = (iota == sel).astype(xv.dtype)
    lane = jax.lax.broadcasted_iota(jnp.int32, (_CB, _N_STATES, n_dims), 2)
    o_ref[...] = jnp.where(lane == 0, onehot[:, :, None], xv)


def kernel(x, W):
    n_chains, n_dims, n_states = x.shape
    xt = jnp.transpose(x, (0, 2, 1))
    w16 = W[:n_states].reshape(1, n_states)
    g = jax.random.gumbel(jax.random.key(42), (n_chains, n_states),
                          dtype=x.dtype)
    ot = pl.pallas_call(
        _gibbs_body,
        grid=(n_chains // _CB,),
        in_specs=[
            pl.BlockSpec((_CB, n_states, n_dims), lambda i: (i, 0, 0)),
            pl.BlockSpec((1, n_states), lambda i: (0, 0)),
            pl.BlockSpec((_CB, n_states), lambda i: (i, 0)),
        ],
        out_specs=pl.BlockSpec((_CB, n_states, n_dims), lambda i: (i, 0, 0)),
        out_shape=jax.ShapeDtypeStruct((n_chains, n_states, n_dims), x.dtype),
    )(xt, w16, g)
    return jnp.transpose(ot, (0, 2, 1))
